# baseline (speedup 1.0000x reference)
import functools
import jax
import jax.numpy as jnp
from jax import lax
from jax.experimental import pallas as pl
from jax.experimental.pallas import tpu as pltpu
from jax.experimental.pallas import tpu_sc as plsc

N = 10000
E = 320000
G = 100
D = 128
NBLK = 10
BN = N // NBLK
EBLK = 160
BE = E // EBLK

NC = 2
NS = 16
NW = NC * NS
EC = 40
EPW = E // NW
NCHUNK = EPW // EC
PCH = 128
NPAD = 10240
NPS = NPAD // NS


def _encode_body(xn_ref, wn_ref, bn_ref, wmf_ref, wmt_ref,
                 x_ref, xf_ref, xt_ref):
    x = jnp.maximum(xn_ref[...] @ wn_ref[...] + bn_ref[...], 0.0)
    x_ref[...] = x
    xf_ref[...] = x @ wmf_ref[...]
    xt_ref[...] = x @ wmt_ref[...]


def _encode(xn, wn, bn, wmf, wmt):
    row = pl.BlockSpec((BN, D), lambda i: (i, 0))
    w = pl.BlockSpec((D, D), lambda i: (0, 0))
    b = pl.BlockSpec((1, D), lambda i: (0, 0))
    out = jax.ShapeDtypeStruct((N, D), jnp.float32)
    return pl.pallas_call(
        _encode_body,
        grid=(NBLK,),
        in_specs=[row, w, b, w, w],
        out_specs=[row, row, row],
        out_shape=[out, out, out],
    )(xn, wn, bn, wmf, wmt)


def _efm_body(ef_ref, we_ref, be_ref, w1_ref, b1_ref, w2_ref, b2_ref,
              w3_ref, b3_ref, o1_ref, o2_ref, o3_ref):
    ef = jnp.maximum(ef_ref[...] @ we_ref[...] + be_ref[...], 0.0)
    o1_ref[...] = ef @ w1_ref[...] + b1_ref[...]
    o2_ref[...] = ef @ w2_ref[...] + b2_ref[...]
    o3_ref[...] = ef @ w3_ref[...] + b3_ref[...]


def _efm(ef, we, be, wms, bms):
    row16 = pl.BlockSpec((BE, 16), lambda i: (i, 0))
    row = pl.BlockSpec((BE, D), lambda i: (i, 0))
    we_s = pl.BlockSpec((16, D), lambda i: (0, 0))
    w = pl.BlockSpec((D, D), lambda i: (0, 0))
    b = pl.BlockSpec((1, D), lambda i: (0, 0))
    out = jax.ShapeDtypeStruct((E, D), jnp.float32)
    return pl.pallas_call(
        _efm_body,
        grid=(EBLK,),
        in_specs=[row16, we_s, b, w, b, w, b, w, b],
        out_specs=[row, row, row],
        out_shape=[out, out, out],
    )(ef, we, be, wms[0], bms[0], wms[1], bms[1], wms[2], bms[2])


def _update_proj_body(x_ref, acc_ref, wux_ref, wua_ref, bu_ref,
                      wmf_ref, wmt_ref, x_out_ref, xf_ref, xt_ref):
    agg = acc_ref[0] + acc_ref[1]
    xn = jnp.maximum(x_ref[...] @ wux_ref[...] + agg @ wua_ref[...]
                     + bu_ref[...], 0.0)
    x_out_ref[...] = xn
    xf_ref[...] = xn @ wmf_ref[...]
    xt_ref[...] = xn @ wmt_ref[...]


def _update_last_body(x_ref, acc_ref, wux_ref, wua_ref, bu_ref, x_out_ref):
    agg = acc_ref[0] + acc_ref[1]
    x_out_ref[...] = jnp.maximum(x_ref[...] @ wux_ref[...]
                                 + agg @ wua_ref[...] + bu_ref[...], 0.0)


def _update(x, acc, wux, wua, bu, wmf=None, wmt=None):
    row = pl.BlockSpec((BN, D), lambda i: (i, 0))
    acc_s = pl.BlockSpec((2, BN, D), lambda i: (0, i, 0))
    w = pl.BlockSpec((D, D), lambda i: (0, 0))
    b = pl.BlockSpec((1, D), lambda i: (0, 0))
    out = jax.ShapeDtypeStruct((N, D), jnp.float32)
    if wmf is None:
        return pl.pallas_call(
            _update_last_body,
            grid=(NBLK,),
            in_specs=[row, acc_s, w, w, b],
            out_specs=row,
            out_shape=out,
        )(x, acc, wux, wua, bu)
    return pl.pallas_call(
        _update_proj_body,
        grid=(NBLK,),
        in_specs=[row, acc_s, w, w, b, w, w],
        out_specs=[row, row, row],
        out_shape=[out, out, out],
    )(x, acc, wux, wua, bu, wmf, wmt)


def _pool_body(x_ref, gid_ref, cnt_ref, wg_ref, bg_ref, out_ref):
    i = pl.program_id(0)
    gids = gid_ref[0]
    rows = lax.broadcasted_iota(jnp.int32, (G, BN), 0)
    p = jnp.where(rows == gids, 1.0, 0.0)
    part = jax.lax.dot_general(p, x_ref[...],
                               (((1,), (0,)), ((), ())),
                               preferred_element_type=jnp.float32)

    @pl.when(i == 0)
    def _():
        out_ref[...] = part

    @pl.when(i > 0)
    def _():
        out_ref[...] = out_ref[...] + part

    @pl.when(i == NBLK - 1)
    def _():
        emb = out_ref[...] / cnt_ref[...]
        out_ref[...] = emb @ wg_ref[...] + bg_ref[...]


def _pool(x, gid3, cnt2, wg, bg):
    row = pl.BlockSpec((BN, D), lambda i: (i, 0))
    gid_s = pl.BlockSpec((1, 1, BN), lambda i: (i, 0, 0))
    cnt_s = pl.BlockSpec((G, 1), lambda i: (0, 0))
    w = pl.BlockSpec((D, D), lambda i: (0, 0))
    b = pl.BlockSpec((1, D), lambda i: (0, 0))
    out_s = pl.BlockSpec((G, D), lambda i: (0, 0))
    return pl.pallas_call(
        _pool_body,
        grid=(NBLK,),
        in_specs=[row, gid_s, cnt_s, w, b],
        out_specs=out_s,
        out_shape=jax.ShapeDtypeStruct((G, D), jnp.float32),
    )(x, gid3, cnt2, wg, bg)


def _edge_body(xf_hbm, xt_hbm, efm_hbm, from_hbm, to_hbm, out_hbm,
               from_buf, to_buf, idxt0, idxt1, rows_f0, rows_f1,
               rows_t0, rows_t1, msg0, msg1, acc, sem0, sem1,
               ssem0, ssem1):
    c = lax.axis_index("c")
    s = lax.axis_index("s")
    w = c * NS + s
    idxt = (idxt0, idxt1)
    rows_f = (rows_f0, rows_f1)
    rows_t = (rows_t0, rows_t1)
    msg = (msg0, msg1)
    sems = (sem0, sem1)
    ssems = (ssem0, ssem1)
    base0 = w * EPW

    pltpu.async_copy(from_hbm.at[w, pl.ds(0, PCH * EC)], from_buf, sem0)
    pltpu.async_copy(to_hbm.at[w, pl.ds(0, PCH * EC)], to_buf, sem1)

    def _zrow(i, carry):
        for j in range(8):
            msg0[i, pl.ds(j * 16, 16)] = jnp.zeros((16,), jnp.float32)
        return carry
    lax.fori_loop(0, EC, _zrow, 0)
    for k in range(NPS // EC):
        pltpu.async_copy(msg0, acc.at[pl.ds(s * NPS + k * EC, EC)], ssem0)
    for k in range(NPS // EC):
        pltpu.make_async_copy(msg0, acc.at[pl.ds(s * NPS + k * EC, EC)],
                              ssem0).wait()
    pltpu.make_async_copy(from_hbm.at[w, pl.ds(0, PCH * EC)], from_buf,
                          sem0).wait()
    pltpu.make_async_copy(to_hbm.at[w, pl.ds(0, PCH * EC)], to_buf,
                          sem1).wait()
    plsc.subcore_barrier()

    def _fire(k, b):
        @pl.when(k >= 2)
        def _():
            pltpu.make_async_copy(msg[b], acc.at[pl.ds(0, EC)],
                                  ssems[b]).wait()
        flat = jnp.bitwise_and(k, PCH - 1) * EC
        for off in range(0, EC - 16, 16):
            idxt[b][pl.ds(off, 16)] = to_buf[pl.ds(flat + off, 16)]
        idxt[b][pl.ds(EC - 16, 16)] = to_buf[pl.ds(flat + EC - 16, 16)]
        pltpu.async_copy(xf_hbm.at[from_buf.at[pl.ds(flat, EC)]],
                         rows_f[b], sems[b])
        pltpu.async_copy(xt_hbm.at[idxt[b]], rows_t[b], sems[b])
        pltpu.async_copy(efm_hbm.at[pl.ds(base0 + k * EC, EC)], msg[b],
                         sems[b])

    def _drain(b):
        pltpu.make_async_copy(xf_hbm.at[pl.ds(0, EC)], rows_f[b],
                              sems[b]).wait()
        pltpu.make_async_copy(xt_hbm.at[pl.ds(0, EC)], rows_t[b],
                              sems[b]).wait()
        pltpu.make_async_copy(efm_hbm.at[pl.ds(0, EC)], msg[b],
                              sems[b]).wait()

    def _process(b):
        def _row(i, cc):
            for j in range(8):
                dsl = pl.ds(j * 16, 16)
                v = rows_f[b][i, dsl] + rows_t[b][i, dsl] + msg[b][i, dsl]
                msg[b][i, dsl] = jnp.maximum(v, 0.0)
            return cc
        lax.fori_loop(0, EC, _row, 0)
        pltpu.async_copy(msg[b], acc.at[idxt[b]], ssems[b], add=True)

    def _mk_pair(limit):
        def _pair(i, carry):
            k1 = 2 * i + 1
            _fire(k1, 1)
            _drain(0)
            _process(0)

            @pl.when(k1 + 1 < limit)
            def _():
                _fire(k1 + 1, 0)
            _drain(1)
            _process(1)
            return carry
        return _pair

    _fire(0, 0)
    lax.fori_loop(0, PCH // 2, _mk_pair(PCH), 0)
    pltpu.sync_copy(from_hbm.at[w, pl.ds(PCH * EC, PCH * EC)], from_buf)
    pltpu.sync_copy(to_hbm.at[w, pl.ds(PCH * EC, PCH * EC)], to_buf)
    _fire(PCH, 0)
    lax.fori_loop(PCH // 2, NCHUNK // 2, _mk_pair(NCHUNK), 0)

    pltpu.make_async_copy(msg[0], acc.at[pl.ds(0, EC)], ssems[0]).wait()
    pltpu.make_async_copy(msg[1], acc.at[pl.ds(0, EC)], ssems[1]).wait()

    plsc.subcore_barrier()
    pltpu.sync_copy(acc.at[pl.ds(s * NPS, NPS)],
                    out_hbm.at[c, pl.ds(s * NPS, NPS)])


@functools.partial(
    pl.kernel,
    mesh=plsc.VectorSubcoreMesh(core_axis_name="c", subcore_axis_name="s"),
    out_type=jax.ShapeDtypeStruct((NC, NPAD, D), jnp.float32),
    scratch_types=[
        pltpu.VMEM((PCH * EC,), jnp.int32),
        pltpu.VMEM((PCH * EC,), jnp.int32),
        pltpu.VMEM((EC,), jnp.int32),
        pltpu.VMEM((EC,), jnp.int32),
        pltpu.VMEM((EC, D), jnp.float32),
        pltpu.VMEM((EC, D), jnp.float32),
        pltpu.VMEM((EC, D), jnp.float32),
        pltpu.VMEM((EC, D), jnp.float32),
        pltpu.VMEM((EC, D), jnp.float32),
        pltpu.VMEM((EC, D), jnp.float32),
        pltpu.VMEM_SHARED((NPAD, D), jnp.float32),
        pltpu.SemaphoreType.DMA,
        pltpu.SemaphoreType.DMA,
        pltpu.SemaphoreType.DMA,
        pltpu.SemaphoreType.DMA,
    ],
)
def _edge_pass(xf_hbm, xt_hbm, efm_hbm, from_hbm, to_hbm, out_hbm,
               from_buf, to_buf, idxt0, idxt1, rows_f0, rows_f1,
               rows_t0, rows_t1, msg0, msg1, acc, sem0, sem1,
               ssem0, ssem1):
    _edge_body(xf_hbm, xt_hbm, efm_hbm, from_hbm, to_hbm, out_hbm,
               from_buf, to_buf, idxt0, idxt1, rows_f0, rows_f1,
               rows_t0, rows_t1, msg0, msg1, acc, sem0, sem1,
               ssem0, ssem1)


def kernel(block_node_features, block_edge_features, block_node_from_idx,
           block_node_to_idx, node_graph_idx, node_num_per_g, params):
    from_idx = jnp.pad(block_node_from_idx.astype(jnp.int32).reshape(NW, EPW),
                       ((0, 0), (0, 2 * PCH * EC - EPW)))
    to_idx = jnp.pad(block_node_to_idx.astype(jnp.int32).reshape(NW, EPW),
                     ((0, 0), (0, 2 * PCH * EC - EPW)))

    wn = params['Wn']
    bn = params['bn'].reshape(1, D)
    we = params['We']
    be = params['be'].reshape(1, D)
    wg = params['Wg']
    bg = params['bg'].reshape(1, D)
    layers = params['layers']
    wmf = [lp['Wm'][:D] for lp in layers]
    wmt = [lp['Wm'][D:2 * D] for lp in layers]
    wme = [lp['Wm'][2 * D:] for lp in layers]
    bm = [lp['bm'].reshape(1, D) for lp in layers]
    wux = [lp['Wu'][:D] for lp in layers]
    wua = [lp['Wu'][D:] for lp in layers]
    bu = [lp['bu'].reshape(1, D) for lp in layers]

    efm = _efm(block_edge_features, we, be, wme, bm)
    x, xf, xt = _encode(block_node_features, wn, bn, wmf[0], wmt[0])
    for l in range(3):
        acc = _edge_pass(xf, xt, efm[l], from_idx, to_idx)[:, :N]
        if l < 2:
            x, xf, xt = _update(x, acc, wux[l], wua[l], bu[l],
                                wmf[l + 1], wmt[l + 1])
        else:
            x = _update(x, acc, wux[l], wua[l], bu[l])

    gid3 = node_graph_idx.astype(jnp.int32).reshape(NBLK, 1, BN)
    cnt2 = node_num_per_g.reshape(G, 1)
    return _pool(x, gid3, cnt2, wg, bg)


# speedup vs baseline: 1.0008x; 1.0008x over previous
import functools
import jax
import jax.numpy as jnp
from jax import lax
from jax.experimental import pallas as pl
from jax.experimental.pallas import tpu as pltpu
from jax.experimental.pallas import tpu_sc as plsc

N = 10000
E = 320000
G = 100
D = 128
NBLK = 10
BN = N // NBLK
EBLK = 160
BE = E // EBLK

NC = 2
NS = 16
NW = NC * NS
EC = 40
EPW = E // NW
NCHUNK = EPW // EC
PCH = 128
NPAD = 10240
NPS = NPAD // NS


def _encode_body(xn_ref, wn_ref, bn_ref, wmf_ref, wmt_ref,
                 x_ref, xf_ref, xt_ref):
    x = jnp.maximum(xn_ref[...] @ wn_ref[...] + bn_ref[...], 0.0)
    x_ref[...] = x
    xf_ref[...] = x @ wmf_ref[...]
    xt_ref[...] = x @ wmt_ref[...]


def _encode(xn, wn, bn, wmf, wmt):
    row = pl.BlockSpec((BN, D), lambda i: (i, 0))
    w = pl.BlockSpec((D, D), lambda i: (0, 0))
    b = pl.BlockSpec((1, D), lambda i: (0, 0))
    out = jax.ShapeDtypeStruct((N, D), jnp.float32)
    return pl.pallas_call(
        _encode_body,
        grid=(NBLK,),
        in_specs=[row, w, b, w, w],
        out_specs=[row, row, row],
        out_shape=[out, out, out],
    )(xn, wn, bn, wmf, wmt)


def _efm_body(ef_ref, we_ref, be_ref, w1_ref, b1_ref, w2_ref, b2_ref,
              w3_ref, b3_ref, o1_ref, o2_ref, o3_ref):
    ef = jnp.maximum(ef_ref[...] @ we_ref[...] + be_ref[...], 0.0)
    o1_ref[...] = ef @ w1_ref[...] + b1_ref[...]
    o2_ref[...] = ef @ w2_ref[...] + b2_ref[...]
    o3_ref[...] = ef @ w3_ref[...] + b3_ref[...]


def _efm(ef, we, be, wms, bms):
    row16 = pl.BlockSpec((BE, 16), lambda i: (i, 0))
    row = pl.BlockSpec((BE, D), lambda i: (i, 0))
    we_s = pl.BlockSpec((16, D), lambda i: (0, 0))
    w = pl.BlockSpec((D, D), lambda i: (0, 0))
    b = pl.BlockSpec((1, D), lambda i: (0, 0))
    out = jax.ShapeDtypeStruct((E, D), jnp.float32)
    return pl.pallas_call(
        _efm_body,
        grid=(EBLK,),
        in_specs=[row16, we_s, b, w, b, w, b, w, b],
        out_specs=[row, row, row],
        out_shape=[out, out, out],
    )(ef, we, be, wms[0], bms[0], wms[1], bms[1], wms[2], bms[2])


def _update_proj_body(x_ref, acc_ref, wux_ref, wua_ref, bu_ref,
                      wmf_ref, wmt_ref, x_out_ref, xf_ref, xt_ref):
    agg = acc_ref[0] + acc_ref[1]
    xn = jnp.maximum(x_ref[...] @ wux_ref[...] + agg @ wua_ref[...]
                     + bu_ref[...], 0.0)
    x_out_ref[...] = xn
    xf_ref[...] = xn @ wmf_ref[...]
    xt_ref[...] = xn @ wmt_ref[...]


def _update_pool_body(x_ref, acc_ref, wux_ref, wua_ref, bu_ref, gid_ref,
                      cnt_ref, wg_ref, bg_ref, out_ref):
    i = pl.program_id(0)
    agg = acc_ref[0] + acc_ref[1]
    xn = jnp.maximum(x_ref[...] @ wux_ref[...] + agg @ wua_ref[...]
                     + bu_ref[...], 0.0)
    gids = gid_ref[0]
    rows = lax.broadcasted_iota(jnp.int32, (G, BN), 0)
    p = jnp.where(rows == gids, 1.0, 0.0)
    part = jax.lax.dot_general(p, xn, (((1,), (0,)), ((), ())),
                               preferred_element_type=jnp.float32)

    @pl.when(i == 0)
    def _():
        out_ref[...] = part

    @pl.when(i > 0)
    def _():
        out_ref[...] = out_ref[...] + part

    @pl.when(i == NBLK - 1)
    def _():
        emb = out_ref[...] / cnt_ref[...]
        out_ref[...] = emb @ wg_ref[...] + bg_ref[...]


def _update_pool(x, acc, wux, wua, bu, gid3, cnt2, wg, bg):
    row = pl.BlockSpec((BN, D), lambda i: (i, 0))
    acc_s = pl.BlockSpec((2, BN, D), lambda i: (0, i, 0))
    w = pl.BlockSpec((D, D), lambda i: (0, 0))
    b = pl.BlockSpec((1, D), lambda i: (0, 0))
    gid_s = pl.BlockSpec((1, 1, BN), lambda i: (i, 0, 0))
    cnt_s = pl.BlockSpec((G, 1), lambda i: (0, 0))
    out_s = pl.BlockSpec((G, D), lambda i: (0, 0))
    return pl.pallas_call(
        _update_pool_body,
        grid=(NBLK,),
        in_specs=[row, acc_s, w, w, b, gid_s, cnt_s, w, b],
        out_specs=out_s,
        out_shape=jax.ShapeDtypeStruct((G, D), jnp.float32),
    )(x, acc, wux, wua, bu, gid3, cnt2, wg, bg)


def _update(x, acc, wux, wua, bu, wmf, wmt):
    row = pl.BlockSpec((BN, D), lambda i: (i, 0))
    acc_s = pl.BlockSpec((2, BN, D), lambda i: (0, i, 0))
    w = pl.BlockSpec((D, D), lambda i: (0, 0))
    b = pl.BlockSpec((1, D), lambda i: (0, 0))
    out = jax.ShapeDtypeStruct((N, D), jnp.float32)
    return pl.pallas_call(
        _update_proj_body,
        grid=(NBLK,),
        in_specs=[row, acc_s, w, w, b, w, w],
        out_specs=[row, row, row],
        out_shape=[out, out, out],
    )(x, acc, wux, wua, bu, wmf, wmt)


def _edge_body(xf_hbm, xt_hbm, efm_hbm, from_hbm, to_hbm, out_hbm,
               from_buf, to_buf, idxt0, idxt1, rows_f0, rows_f1,
               rows_t0, rows_t1, msg0, msg1, acc, sem0, sem1,
               ssem0, ssem1):
    c = lax.axis_index("c")
    s = lax.axis_index("s")
    w = c * NS + s
    idxt = (idxt0, idxt1)
    rows_f = (rows_f0, rows_f1)
    rows_t = (rows_t0, rows_t1)
    msg = (msg0, msg1)
    sems = (sem0, sem1)
    ssems = (ssem0, ssem1)
    base0 = w * EPW

    pltpu.async_copy(from_hbm.at[w, pl.ds(0, PCH * EC)], from_buf, sem0)
    pltpu.async_copy(to_hbm.at[w, pl.ds(0, PCH * EC)], to_buf, sem1)

    def _zrow(i, carry):
        for j in range(8):
            msg0[i, pl.ds(j * 16, 16)] = jnp.zeros((16,), jnp.float32)
        return carry
    lax.fori_loop(0, EC, _zrow, 0)
    for k in range(NPS // EC):
        pltpu.async_copy(msg0, acc.at[pl.ds(s * NPS + k * EC, EC)], ssem0)
    for k in range(NPS // EC):
        pltpu.make_async_copy(msg0, acc.at[pl.ds(s * NPS + k * EC, EC)],
                              ssem0).wait()
    pltpu.make_async_copy(from_hbm.at[w, pl.ds(0, PCH * EC)], from_buf,
                          sem0).wait()
    pltpu.make_async_copy(to_hbm.at[w, pl.ds(0, PCH * EC)], to_buf,
                          sem1).wait()
    plsc.subcore_barrier()

    def _fire(k, b):
        @pl.when(k >= 2)
        def _():
            pltpu.make_async_copy(msg[b], acc.at[pl.ds(0, EC)],
                                  ssems[b]).wait()
        flat = jnp.bitwise_and(k, PCH - 1) * EC
        for off in range(0, EC - 16, 16):
            idxt[b][pl.ds(off, 16)] = to_buf[pl.ds(flat + off, 16)]
        idxt[b][pl.ds(EC - 16, 16)] = to_buf[pl.ds(flat + EC - 16, 16)]
        pltpu.async_copy(xf_hbm.at[from_buf.at[pl.ds(flat, EC)]],
                         rows_f[b], sems[b])
        pltpu.async_copy(xt_hbm.at[idxt[b]], rows_t[b], sems[b])
        pltpu.async_copy(efm_hbm.at[pl.ds(base0 + k * EC, EC)], msg[b],
                         sems[b])

    def _drain(b):
        pltpu.make_async_copy(xf_hbm.at[pl.ds(0, EC)], rows_f[b],
                              sems[b]).wait()
        pltpu.make_async_copy(xt_hbm.at[pl.ds(0, EC)], rows_t[b],
                              sems[b]).wait()
        pltpu.make_async_copy(efm_hbm.at[pl.ds(0, EC)], msg[b],
                              sems[b]).wait()

    def _process(b):
        def _row4(i4, cc):
            for r in range(4):
                i = i4 * 4 + r
                for j in range(8):
                    dsl = pl.ds(j * 16, 16)
                    v = (rows_f[b][i, dsl] + rows_t[b][i, dsl]
                         + msg[b][i, dsl])
                    msg[b][i, dsl] = jnp.maximum(v, 0.0)
            return cc
        lax.fori_loop(0, EC // 4, _row4, 0)
        pltpu.async_copy(msg[b], acc.at[idxt[b]], ssems[b], add=True)

    def _mk_pair(limit):
        def _pair(i, carry):
            k1 = 2 * i + 1
            _fire(k1, 1)
            _drain(0)
            _process(0)

            @pl.when(k1 + 1 < limit)
            def _():
                _fire(k1 + 1, 0)
            _drain(1)
            _process(1)
            return carry
        return _pair

    _fire(0, 0)
    lax.fori_loop(0, PCH // 2, _mk_pair(PCH), 0)
    pltpu.sync_copy(from_hbm.at[w, pl.ds(PCH * EC, PCH * EC)], from_buf)
    pltpu.sync_copy(to_hbm.at[w, pl.ds(PCH * EC, PCH * EC)], to_buf)
    _fire(PCH, 0)
    lax.fori_loop(PCH // 2, NCHUNK // 2, _mk_pair(NCHUNK), 0)

    pltpu.make_async_copy(msg[0], acc.at[pl.ds(0, EC)], ssems[0]).wait()
    pltpu.make_async_copy(msg[1], acc.at[pl.ds(0, EC)], ssems[1]).wait()

    plsc.subcore_barrier()
    pltpu.sync_copy(acc.at[pl.ds(s * NPS, NPS)],
                    out_hbm.at[c, pl.ds(s * NPS, NPS)])


@functools.partial(
    pl.kernel,
    mesh=plsc.VectorSubcoreMesh(core_axis_name="c", subcore_axis_name="s"),
    out_type=jax.ShapeDtypeStruct((NC, NPAD, D), jnp.float32),
    scratch_types=[
        pltpu.VMEM((PCH * EC,), jnp.int32),
        pltpu.VMEM((PCH * EC,), jnp.int32),
        pltpu.VMEM((EC,), jnp.int32),
        pltpu.VMEM((EC,), jnp.int32),
        pltpu.VMEM((EC, D), jnp.float32),
        pltpu.VMEM((EC, D), jnp.float32),
        pltpu.VMEM((EC, D), jnp.float32),
        pltpu.VMEM((EC, D), jnp.float32),
        pltpu.VMEM((EC, D), jnp.float32),
        pltpu.VMEM((EC, D), jnp.float32),
        pltpu.VMEM_SHARED((NPAD, D), jnp.float32),
        pltpu.SemaphoreType.DMA,
        pltpu.SemaphoreType.DMA,
        pltpu.SemaphoreType.DMA,
        pltpu.SemaphoreType.DMA,
    ],
)
def _edge_pass(xf_hbm, xt_hbm, efm_hbm, from_hbm, to_hbm, out_hbm,
               from_buf, to_buf, idxt0, idxt1, rows_f0, rows_f1,
               rows_t0, rows_t1, msg0, msg1, acc, sem0, sem1,
               ssem0, ssem1):
    _edge_body(xf_hbm, xt_hbm, efm_hbm, from_hbm, to_hbm, out_hbm,
               from_buf, to_buf, idxt0, idxt1, rows_f0, rows_f1,
               rows_t0, rows_t1, msg0, msg1, acc, sem0, sem1,
               ssem0, ssem1)


def kernel(block_node_features, block_edge_features, block_node_from_idx,
           block_node_to_idx, node_graph_idx, node_num_per_g, params):
    from_idx = jnp.pad(block_node_from_idx.astype(jnp.int32).reshape(NW, EPW),
                       ((0, 0), (0, 2 * PCH * EC - EPW)))
    to_idx = jnp.pad(block_node_to_idx.astype(jnp.int32).reshape(NW, EPW),
                     ((0, 0), (0, 2 * PCH * EC - EPW)))

    wn = params['Wn']
    bn = params['bn'].reshape(1, D)
    we = params['We']
    be = params['be'].reshape(1, D)
    wg = params['Wg']
    bg = params['bg'].reshape(1, D)
    layers = params['layers']
    wmf = [lp['Wm'][:D] for lp in layers]
    wmt = [lp['Wm'][D:2 * D] for lp in layers]
    wme = [lp['Wm'][2 * D:] for lp in layers]
    bm = [lp['bm'].reshape(1, D) for lp in layers]
    wux = [lp['Wu'][:D] for lp in layers]
    wua = [lp['Wu'][D:] for lp in layers]
    bu = [lp['bu'].reshape(1, D) for lp in layers]

    gid3 = node_graph_idx.astype(jnp.int32).reshape(NBLK, 1, BN)
    cnt2 = node_num_per_g.reshape(G, 1)

    efm = _efm(block_edge_features, we, be, wme, bm)
    x, xf, xt = _encode(block_node_features, wn, bn, wmf[0], wmt[0])
    for l in range(2):
        acc = _edge_pass(xf, xt, efm[l], from_idx, to_idx)[:, :N]
        x, xf, xt = _update(x, acc, wux[l], wua[l], bu[l],
                            wmf[l + 1], wmt[l + 1])
    acc = _edge_pass(xf, xt, efm[2], from_idx, to_idx)[:, :N]
    return _update_pool(x, acc, wux[2], wua[2], bu[2], gid3, cnt2, wg, bg)


# speedup vs baseline: 1.0156x; 1.0147x over previous
import functools
import jax
import jax.numpy as jnp
from jax import lax
from jax.experimental import pallas as pl
from jax.experimental.pallas import tpu as pltpu
from jax.experimental.pallas import tpu_sc as plsc

N = 10000
E = 320000
G = 100
D = 128
NBLK = 10
BN = N // NBLK
EBLK = 160
BE = E // EBLK

NC = 2
NS = 16
NW = NC * NS
EC = 40
EPW = E // NW
NCHUNK = EPW // EC
PCH = 128
NPAD = 10240
NPS = NPAD // NS


def _encode_body(xn_ref, wn_ref, bn_ref, wmf_ref, wmt_ref,
                 x_ref, xf_ref, xt_ref):
    x = jnp.maximum(xn_ref[...] @ wn_ref[...] + bn_ref[...], 0.0)
    x_ref[...] = x
    xf_ref[...] = x @ wmf_ref[...]
    xt_ref[...] = x @ wmt_ref[...]


def _encode(xn, wn, bn, wmf, wmt):
    row = pl.BlockSpec((BN, D), lambda i: (i, 0))
    w = pl.BlockSpec((D, D), lambda i: (0, 0))
    b = pl.BlockSpec((1, D), lambda i: (0, 0))
    out = jax.ShapeDtypeStruct((N, D), jnp.float32)
    return pl.pallas_call(
        _encode_body,
        grid=(NBLK,),
        in_specs=[row, w, b, w, w],
        out_specs=[row, row, row],
        out_shape=[out, out, out],
    )(xn, wn, bn, wmf, wmt)


def _efm_body(ef_ref, we_ref, be_ref, w1_ref, b1_ref, w2_ref, b2_ref,
              w3_ref, b3_ref, o1_ref, o2_ref, o3_ref):
    ef = jnp.maximum(ef_ref[...] @ we_ref[...] + be_ref[...], 0.0)
    o1_ref[...] = ef @ w1_ref[...] + b1_ref[...]
    o2_ref[...] = ef @ w2_ref[...] + b2_ref[...]
    o3_ref[...] = ef @ w3_ref[...] + b3_ref[...]


def _efm(ef, we, be, wms, bms):
    row16 = pl.BlockSpec((BE, 16), lambda i: (i, 0))
    row = pl.BlockSpec((BE, D), lambda i: (i, 0))
    we_s = pl.BlockSpec((16, D), lambda i: (0, 0))
    w = pl.BlockSpec((D, D), lambda i: (0, 0))
    b = pl.BlockSpec((1, D), lambda i: (0, 0))
    out = jax.ShapeDtypeStruct((E, D), jnp.float32)
    return pl.pallas_call(
        _efm_body,
        grid=(EBLK,),
        in_specs=[row16, we_s, b, w, b, w, b, w, b],
        out_specs=[row, row, row],
        out_shape=[out, out, out],
    )(ef, we, be, wms[0], bms[0], wms[1], bms[1], wms[2], bms[2])


def _update_proj_body(x_ref, acc_ref, wux_ref, wua_ref, bu_ref,
                      wmf_ref, wmt_ref, x_out_ref, xf_ref, xt_ref):
    agg = acc_ref[0] + acc_ref[1]
    xn = jnp.maximum(x_ref[...] @ wux_ref[...] + agg @ wua_ref[...]
                     + bu_ref[...], 0.0)
    x_out_ref[...] = xn
    xf_ref[...] = xn @ wmf_ref[...]
    xt_ref[...] = xn @ wmt_ref[...]


def _update_pool_body(x_ref, acc_ref, wux_ref, wua_ref, bu_ref, gid_ref,
                      cnt_ref, wg_ref, bg_ref, out_ref):
    i = pl.program_id(0)
    agg = acc_ref[0] + acc_ref[1]
    xn = jnp.maximum(x_ref[...] @ wux_ref[...] + agg @ wua_ref[...]
                     + bu_ref[...], 0.0)
    gids = gid_ref[0]
    rows = lax.broadcasted_iota(jnp.int32, (G, BN), 0)
    p = jnp.where(rows == gids, 1.0, 0.0)
    part = jax.lax.dot_general(p, xn, (((1,), (0,)), ((), ())),
                               preferred_element_type=jnp.float32)

    @pl.when(i == 0)
    def _():
        out_ref[...] = part

    @pl.when(i > 0)
    def _():
        out_ref[...] = out_ref[...] + part

    @pl.when(i == NBLK - 1)
    def _():
        emb = out_ref[...] / cnt_ref[...]
        out_ref[...] = emb @ wg_ref[...] + bg_ref[...]


def _update_pool(x, acc, wux, wua, bu, gid3, cnt2, wg, bg):
    row = pl.BlockSpec((BN, D), lambda i: (i, 0))
    acc_s = pl.BlockSpec((2, BN, D), lambda i: (0, i, 0))
    w = pl.BlockSpec((D, D), lambda i: (0, 0))
    b = pl.BlockSpec((1, D), lambda i: (0, 0))
    gid_s = pl.BlockSpec((1, 1, BN), lambda i: (i, 0, 0))
    cnt_s = pl.BlockSpec((G, 1), lambda i: (0, 0))
    out_s = pl.BlockSpec((G, D), lambda i: (0, 0))
    return pl.pallas_call(
        _update_pool_body,
        grid=(NBLK,),
        in_specs=[row, acc_s, w, w, b, gid_s, cnt_s, w, b],
        out_specs=out_s,
        out_shape=jax.ShapeDtypeStruct((G, D), jnp.float32),
    )(x, acc, wux, wua, bu, gid3, cnt2, wg, bg)


def _update(x, acc, wux, wua, bu, wmf, wmt):
    row = pl.BlockSpec((BN, D), lambda i: (i, 0))
    acc_s = pl.BlockSpec((2, BN, D), lambda i: (0, i, 0))
    w = pl.BlockSpec((D, D), lambda i: (0, 0))
    b = pl.BlockSpec((1, D), lambda i: (0, 0))
    out = jax.ShapeDtypeStruct((N, D), jnp.float32)
    return pl.pallas_call(
        _update_proj_body,
        grid=(NBLK,),
        in_specs=[row, acc_s, w, w, b, w, w],
        out_specs=[row, row, row],
        out_shape=[out, out, out],
    )(x, acc, wux, wua, bu, wmf, wmt)


def _edge_body(xf_hbm, xt_hbm, efm_hbm, from_hbm, to_hbm, out_hbm,
               from_buf, to_buf, idxt0, idxt1, rows_f0, rows_f1,
               rows_t0, rows_t1, msg0, msg1, acc, sem0, sem1,
               ssem0, ssem1):
    c = lax.axis_index("c")
    s = lax.axis_index("s")
    w = c * NS + s
    idxt = (idxt0, idxt1)
    rows_f = (rows_f0, rows_f1)
    rows_t = (rows_t0, rows_t1)
    msg = (msg0, msg1)
    sems = (sem0, sem1)
    ssems = (ssem0, ssem1)
    base0 = w * EPW

    pltpu.async_copy(from_hbm.at[w, pl.ds(0, PCH * EC)], from_buf, sem0)
    pltpu.async_copy(to_hbm.at[w, pl.ds(0, PCH * EC)], to_buf, sem1)

    def _zrow(i, carry):
        for j in range(8):
            msg0[i, pl.ds(j * 16, 16)] = jnp.zeros((16,), jnp.float32)
        return carry
    lax.fori_loop(0, EC, _zrow, 0)
    for k in range(NPS // EC):
        pltpu.async_copy(msg0, acc.at[pl.ds(s * NPS + k * EC, EC)], ssem0)
    for k in range(NPS // EC):
        pltpu.make_async_copy(msg0, acc.at[pl.ds(s * NPS + k * EC, EC)],
                              ssem0).wait()
    pltpu.make_async_copy(from_hbm.at[w, pl.ds(0, PCH * EC)], from_buf,
                          sem0).wait()
    pltpu.make_async_copy(to_hbm.at[w, pl.ds(0, PCH * EC)], to_buf,
                          sem1).wait()
    plsc.subcore_barrier()

    def _fire(k, b):
        @pl.when(k >= 2)
        def _():
            pltpu.make_async_copy(msg[b], acc.at[pl.ds(0, EC)],
                                  ssems[b]).wait()
        flat = jnp.bitwise_and(k, PCH - 1) * EC
        for off in range(0, EC - 16, 16):
            idxt[b][pl.ds(off, 16)] = to_buf[pl.ds(flat + off, 16)]
        idxt[b][pl.ds(EC - 16, 16)] = to_buf[pl.ds(flat + EC - 16, 16)]
        pltpu.async_copy(xf_hbm.at[from_buf.at[pl.ds(flat, EC)]],
                         rows_f[b], sems[b])
        pltpu.async_copy(xt_hbm.at[idxt[b]], rows_t[b], sems[b])
        pltpu.async_copy(efm_hbm.at[pl.ds(base0 + k * EC, EC)], msg[b],
                         sems[b])

    def _drain(b):
        pltpu.make_async_copy(xf_hbm.at[pl.ds(0, EC)], rows_f[b],
                              sems[b]).wait()
        pltpu.make_async_copy(xt_hbm.at[pl.ds(0, EC)], rows_t[b],
                              sems[b]).wait()
        pltpu.make_async_copy(efm_hbm.at[pl.ds(0, EC)], msg[b],
                              sems[b]).wait()

    def _process(b):
        def _row4(i4, cc):
            for r in range(4):
                i = i4 * 4 + r
                for j in range(8):
                    dsl = pl.ds(j * 16, 16)
                    v = (rows_f[b][i, dsl] + rows_t[b][i, dsl]
                         + msg[b][i, dsl])
                    msg[b][i, dsl] = jnp.maximum(v, 0.0)
            return cc
        lax.fori_loop(0, EC // 4, _row4, 0)
        pltpu.async_copy(msg[b], acc.at[idxt[b]], ssems[b], add=True)

    def _mk_pair(limit):
        def _pair(i, carry):
            k1 = 2 * i + 1
            _fire(k1, 1)
            _drain(0)
            _process(0)

            @pl.when(k1 + 1 < limit)
            def _():
                _fire(k1 + 1, 0)
            _drain(1)
            _process(1)
            return carry
        return _pair

    _fire(0, 0)
    lax.fori_loop(0, PCH // 2, _mk_pair(PCH), 0)
    pltpu.sync_copy(from_hbm.at[w, pl.ds(PCH * EC, PCH * EC)], from_buf)
    pltpu.sync_copy(to_hbm.at[w, pl.ds(PCH * EC, PCH * EC)], to_buf)
    _fire(PCH, 0)
    lax.fori_loop(PCH // 2, NCHUNK // 2, _mk_pair(NCHUNK), 0)

    pltpu.make_async_copy(msg[0], acc.at[pl.ds(0, EC)], ssems[0]).wait()
    pltpu.make_async_copy(msg[1], acc.at[pl.ds(0, EC)], ssems[1]).wait()

    plsc.subcore_barrier()
    pltpu.sync_copy(acc.at[pl.ds(s * NPS, NPS)],
                    out_hbm.at[c, pl.ds(s * NPS, NPS)])


@functools.partial(
    pl.kernel,
    mesh=plsc.VectorSubcoreMesh(core_axis_name="c", subcore_axis_name="s"),
    out_type=jax.ShapeDtypeStruct((NC, NPAD, D), jnp.float32),
    scratch_types=[
        pltpu.VMEM((PCH * EC,), jnp.int32),
        pltpu.VMEM((PCH * EC,), jnp.int32),
        pltpu.VMEM((EC,), jnp.int32),
        pltpu.VMEM((EC,), jnp.int32),
        pltpu.VMEM((EC, D), jnp.float32),
        pltpu.VMEM((EC, D), jnp.float32),
        pltpu.VMEM((EC, D), jnp.float32),
        pltpu.VMEM((EC, D), jnp.float32),
        pltpu.VMEM((EC, D), jnp.float32),
        pltpu.VMEM((EC, D), jnp.float32),
        pltpu.VMEM_SHARED((NPAD, D), jnp.float32),
        pltpu.SemaphoreType.DMA,
        pltpu.SemaphoreType.DMA,
        pltpu.SemaphoreType.DMA,
        pltpu.SemaphoreType.DMA,
    ],
)
def _edge_pass(xf_hbm, xt_hbm, efm_hbm, from_hbm, to_hbm, out_hbm,
               from_buf, to_buf, idxt0, idxt1, rows_f0, rows_f1,
               rows_t0, rows_t1, msg0, msg1, acc, sem0, sem1,
               ssem0, ssem1):
    _edge_body(xf_hbm, xt_hbm, efm_hbm, from_hbm, to_hbm, out_hbm,
               from_buf, to_buf, idxt0, idxt1, rows_f0, rows_f1,
               rows_t0, rows_t1, msg0, msg1, acc, sem0, sem1,
               ssem0, ssem1)


def kernel(block_node_features, block_edge_features, block_node_from_idx,
           block_node_to_idx, node_graph_idx, node_num_per_g, params):
    from_idx = jnp.pad(block_node_from_idx.astype(jnp.int32).reshape(NW, EPW),
                       ((0, 0), (0, 2 * PCH * EC - EPW)))
    to_idx = jnp.pad(block_node_to_idx.astype(jnp.int32).reshape(NW, EPW),
                     ((0, 0), (0, 2 * PCH * EC - EPW)))

    wn = params['Wn']
    bn = params['bn'].reshape(1, D)
    we = params['We']
    be = params['be'].reshape(1, D)
    wg = params['Wg']
    bg = params['bg'].reshape(1, D)
    layers = params['layers']
    wmf = [lp['Wm'][:D] for lp in layers]
    wmt = [lp['Wm'][D:2 * D] for lp in layers]
    wme = [lp['Wm'][2 * D:] for lp in layers]
    bm = [lp['bm'].reshape(1, D) for lp in layers]
    wux = [lp['Wu'][:D] for lp in layers]
    wua = [lp['Wu'][D:] for lp in layers]
    bu = [lp['bu'].reshape(1, D) for lp in layers]

    gid3 = node_graph_idx.astype(jnp.int32).reshape(NBLK, 1, BN)
    cnt2 = node_num_per_g.reshape(G, 1)

    efm = _efm(block_edge_features, we, be, wme, bm)
    x, xf, xt = _encode(block_node_features, wn, bn, wmf[0], wmt[0])
    for l in range(2):
        acc = _edge_pass(xf, xt, efm[l], from_idx, to_idx)
        x, xf, xt = _update(x, acc, wux[l], wua[l], bu[l],
                            wmf[l + 1], wmt[l + 1])
    acc = _edge_pass(xf, xt, efm[2], from_idx, to_idx)
    return _update_pool(x, acc, wux[2], wua[2], bu[2], gid3, cnt2, wg, bg)


# speedup vs baseline: 1.1468x; 1.1292x over previous
import functools
import jax
import jax.numpy as jnp
from jax import lax
from jax.experimental import pallas as pl
from jax.experimental.pallas import tpu as pltpu
from jax.experimental.pallas import tpu_sc as plsc

N = 10000
E = 320000
G = 100
D = 128
NBLK = 10
BN = N // NBLK
EBLK = 125
BE = E // EBLK

NC = 2
NS = 16
NW = NC * NS
EC = 40
EPW = E // NW
NCHUNK = EPW // EC
PCH = 128
NPAD = 10240
NPS = NPAD // NS


def _encode_body(xn_ref, wn_ref, bn_ref, wmf_ref, wmt_ref,
                 x_ref, xf_ref, xt_ref):
    x = jnp.maximum(xn_ref[...] @ wn_ref[...] + bn_ref[...], 0.0)
    x_ref[...] = x
    xf_ref[...] = x @ wmf_ref[...]
    xt_ref[...] = x @ wmt_ref[...]


def _encode(xn, wn, bn, wmf, wmt):
    row = pl.BlockSpec((BN, D), lambda i: (i, 0))
    w = pl.BlockSpec((D, D), lambda i: (0, 0))
    b = pl.BlockSpec((1, D), lambda i: (0, 0))
    out = jax.ShapeDtypeStruct((N, D), jnp.float32)
    return pl.pallas_call(
        _encode_body,
        grid=(NBLK,),
        in_specs=[row, w, b, w, w],
        out_specs=[row, row, row],
        out_shape=[out, out, out],
    )(xn, wn, bn, wmf, wmt)


def _efm_body(eft_ref, we_ref, be_ref, w1_ref, b1_ref, w2_ref, b2_ref,
              w3_ref, b3_ref, o1_ref, o2_ref, o3_ref):
    prod = jax.lax.dot_general(eft_ref[...], we_ref[...],
                               (((0,), (0,)), ((), ())),
                               preferred_element_type=jnp.float32)
    ef = jnp.maximum(prod + be_ref[...], 0.0)
    o1_ref[...] = ef @ w1_ref[...] + b1_ref[...]
    o2_ref[...] = ef @ w2_ref[...] + b2_ref[...]
    o3_ref[...] = ef @ w3_ref[...] + b3_ref[...]


def _efm(eft, we, be, wms, bms):
    col16 = pl.BlockSpec((16, BE), lambda i: (0, i))
    row = pl.BlockSpec((BE, D), lambda i: (i, 0))
    we_s = pl.BlockSpec((16, D), lambda i: (0, 0))
    w = pl.BlockSpec((D, D), lambda i: (0, 0))
    b = pl.BlockSpec((1, D), lambda i: (0, 0))
    out = jax.ShapeDtypeStruct((E, D), jnp.float32)
    return pl.pallas_call(
        _efm_body,
        grid=(EBLK,),
        in_specs=[col16, we_s, b, w, b, w, b, w, b],
        out_specs=[row, row, row],
        out_shape=[out, out, out],
    )(eft, we, be, wms[0], bms[0], wms[1], bms[1], wms[2], bms[2])


def _update_proj_body(x_ref, acc_ref, wux_ref, wua_ref, bu_ref,
                      wmf_ref, wmt_ref, x_out_ref, xf_ref, xt_ref):
    agg = acc_ref[0] + acc_ref[1]
    xn = jnp.maximum(x_ref[...] @ wux_ref[...] + agg @ wua_ref[...]
                     + bu_ref[...], 0.0)
    x_out_ref[...] = xn
    xf_ref[...] = xn @ wmf_ref[...]
    xt_ref[...] = xn @ wmt_ref[...]


def _update_pool_body(x_ref, acc_ref, wux_ref, wua_ref, bu_ref, gid_ref,
                      cnt_ref, wg_ref, bg_ref, out_ref):
    i = pl.program_id(0)
    agg = acc_ref[0] + acc_ref[1]
    xn = jnp.maximum(x_ref[...] @ wux_ref[...] + agg @ wua_ref[...]
                     + bu_ref[...], 0.0)
    gids = gid_ref[0]
    rows = lax.broadcasted_iota(jnp.int32, (G, BN), 0)
    p = jnp.where(rows == gids, 1.0, 0.0)
    part = jax.lax.dot_general(p, xn, (((1,), (0,)), ((), ())),
                               preferred_element_type=jnp.float32)

    @pl.when(i == 0)
    def _():
        out_ref[...] = part

    @pl.when(i > 0)
    def _():
        out_ref[...] = out_ref[...] + part

    @pl.when(i == NBLK - 1)
    def _():
        emb = out_ref[...] / cnt_ref[...]
        out_ref[...] = emb @ wg_ref[...] + bg_ref[...]


def _update_pool(x, acc, wux, wua, bu, gid3, cnt2, wg, bg):
    row = pl.BlockSpec((BN, D), lambda i: (i, 0))
    acc_s = pl.BlockSpec((2, BN, D), lambda i: (0, i, 0))
    w = pl.BlockSpec((D, D), lambda i: (0, 0))
    b = pl.BlockSpec((1, D), lambda i: (0, 0))
    gid_s = pl.BlockSpec((1, 1, BN), lambda i: (i, 0, 0))
    cnt_s = pl.BlockSpec((G, 1), lambda i: (0, 0))
    out_s = pl.BlockSpec((G, D), lambda i: (0, 0))
    return pl.pallas_call(
        _update_pool_body,
        grid=(NBLK,),
        in_specs=[row, acc_s, w, w, b, gid_s, cnt_s, w, b],
        out_specs=out_s,
        out_shape=jax.ShapeDtypeStruct((G, D), jnp.float32),
    )(x, acc, wux, wua, bu, gid3, cnt2, wg, bg)


def _update(x, acc, wux, wua, bu, wmf, wmt):
    row = pl.BlockSpec((BN, D), lambda i: (i, 0))
    acc_s = pl.BlockSpec((2, BN, D), lambda i: (0, i, 0))
    w = pl.BlockSpec((D, D), lambda i: (0, 0))
    b = pl.BlockSpec((1, D), lambda i: (0, 0))
    out = jax.ShapeDtypeStruct((N, D), jnp.float32)
    return pl.pallas_call(
        _update_proj_body,
        grid=(NBLK,),
        in_specs=[row, acc_s, w, w, b, w, w],
        out_specs=[row, row, row],
        out_shape=[out, out, out],
    )(x, acc, wux, wua, bu, wmf, wmt)


def _edge_body(xf_hbm, xt_hbm, efm_hbm, from_hbm, to_hbm, out_hbm,
               from_buf, to_buf, idxt0, idxt1, rows_f0, rows_f1,
               rows_t0, rows_t1, msg0, msg1, acc, sem0, sem1,
               ssem0, ssem1):
    c = lax.axis_index("c")
    s = lax.axis_index("s")
    w = c * NS + s
    idxt = (idxt0, idxt1)
    rows_f = (rows_f0, rows_f1)
    rows_t = (rows_t0, rows_t1)
    msg = (msg0, msg1)
    sems = (sem0, sem1)
    ssems = (ssem0, ssem1)
    base0 = w * EPW

    pltpu.async_copy(from_hbm.at[w, pl.ds(0, PCH * EC)], from_buf, sem0)
    pltpu.async_copy(to_hbm.at[w, pl.ds(0, PCH * EC)], to_buf, sem1)

    def _zrow(i, carry):
        for j in range(8):
            msg0[i, pl.ds(j * 16, 16)] = jnp.zeros((16,), jnp.float32)
        return carry
    lax.fori_loop(0, EC, _zrow, 0)
    for k in range(NPS // EC):
        pltpu.async_copy(msg0, acc.at[pl.ds(s * NPS + k * EC, EC)], ssem0)
    for k in range(NPS // EC):
        pltpu.make_async_copy(msg0, acc.at[pl.ds(s * NPS + k * EC, EC)],
                              ssem0).wait()
    pltpu.make_async_copy(from_hbm.at[w, pl.ds(0, PCH * EC)], from_buf,
                          sem0).wait()
    pltpu.make_async_copy(to_hbm.at[w, pl.ds(0, PCH * EC)], to_buf,
                          sem1).wait()
    plsc.subcore_barrier()

    def _fire(k, b):
        @pl.when(k >= 2)
        def _():
            pltpu.make_async_copy(msg[b], acc.at[pl.ds(0, EC)],
                                  ssems[b]).wait()
        flat = jnp.bitwise_and(k, PCH - 1) * EC
        for off in range(0, EC - 16, 16):
            idxt[b][pl.ds(off, 16)] = to_buf[pl.ds(flat + off, 16)]
        idxt[b][pl.ds(EC - 16, 16)] = to_buf[pl.ds(flat + EC - 16, 16)]
        pltpu.async_copy(xf_hbm.at[from_buf.at[pl.ds(flat, EC)]],
                         rows_f[b], sems[b])
        pltpu.async_copy(xt_hbm.at[idxt[b]], rows_t[b], sems[b])
        pltpu.async_copy(efm_hbm.at[pl.ds(base0 + k * EC, EC)], msg[b],
                         sems[b])

    def _drain(b):
        pltpu.make_async_copy(xf_hbm.at[pl.ds(0, EC)], rows_f[b],
                              sems[b]).wait()
        pltpu.make_async_copy(xt_hbm.at[pl.ds(0, EC)], rows_t[b],
                              sems[b]).wait()
        pltpu.make_async_copy(efm_hbm.at[pl.ds(0, EC)], msg[b],
                              sems[b]).wait()

    def _process(b):
        def _row4(i4, cc):
            for r in range(4):
                i = i4 * 4 + r
                for j in range(8):
                    dsl = pl.ds(j * 16, 16)
                    v = (rows_f[b][i, dsl] + rows_t[b][i, dsl]
                         + msg[b][i, dsl])
                    msg[b][i, dsl] = jnp.maximum(v, 0.0)
            return cc
        lax.fori_loop(0, EC // 4, _row4, 0)
        pltpu.async_copy(msg[b], acc.at[idxt[b]], ssems[b], add=True)

    def _mk_pair(limit):
        def _pair(i, carry):
            k1 = 2 * i + 1
            _fire(k1, 1)
            _drain(0)
            _process(0)

            @pl.when(k1 + 1 < limit)
            def _():
                _fire(k1 + 1, 0)
            _drain(1)
            _process(1)
            return carry
        return _pair

    _fire(0, 0)
    lax.fori_loop(0, PCH // 2, _mk_pair(PCH), 0)
    pltpu.sync_copy(from_hbm.at[w, pl.ds(PCH * EC, PCH * EC)], from_buf)
    pltpu.sync_copy(to_hbm.at[w, pl.ds(PCH * EC, PCH * EC)], to_buf)
    _fire(PCH, 0)
    lax.fori_loop(PCH // 2, NCHUNK // 2, _mk_pair(NCHUNK), 0)

    pltpu.make_async_copy(msg[0], acc.at[pl.ds(0, EC)], ssems[0]).wait()
    pltpu.make_async_copy(msg[1], acc.at[pl.ds(0, EC)], ssems[1]).wait()

    plsc.subcore_barrier()
    pltpu.sync_copy(acc.at[pl.ds(s * NPS, NPS)],
                    out_hbm.at[c, pl.ds(s * NPS, NPS)])


@functools.partial(
    pl.kernel,
    mesh=plsc.VectorSubcoreMesh(core_axis_name="c", subcore_axis_name="s"),
    out_type=jax.ShapeDtypeStruct((NC, NPAD, D), jnp.float32),
    scratch_types=[
        pltpu.VMEM((PCH * EC,), jnp.int32),
        pltpu.VMEM((PCH * EC,), jnp.int32),
        pltpu.VMEM((EC,), jnp.int32),
        pltpu.VMEM((EC,), jnp.int32),
        pltpu.VMEM((EC, D), jnp.float32),
        pltpu.VMEM((EC, D), jnp.float32),
        pltpu.VMEM((EC, D), jnp.float32),
        pltpu.VMEM((EC, D), jnp.float32),
        pltpu.VMEM((EC, D), jnp.float32),
        pltpu.VMEM((EC, D), jnp.float32),
        pltpu.VMEM_SHARED((NPAD, D), jnp.float32),
        pltpu.SemaphoreType.DMA,
        pltpu.SemaphoreType.DMA,
        pltpu.SemaphoreType.DMA,
        pltpu.SemaphoreType.DMA,
    ],
)
def _edge_pass(xf_hbm, xt_hbm, efm_hbm, from_hbm, to_hbm, out_hbm,
               from_buf, to_buf, idxt0, idxt1, rows_f0, rows_f1,
               rows_t0, rows_t1, msg0, msg1, acc, sem0, sem1,
               ssem0, ssem1):
    _edge_body(xf_hbm, xt_hbm, efm_hbm, from_hbm, to_hbm, out_hbm,
               from_buf, to_buf, idxt0, idxt1, rows_f0, rows_f1,
               rows_t0, rows_t1, msg0, msg1, acc, sem0, sem1,
               ssem0, ssem1)


def kernel(block_node_features, block_edge_features, block_node_from_idx,
           block_node_to_idx, node_graph_idx, node_num_per_g, params):
    from_idx = jnp.pad(block_node_from_idx.astype(jnp.int32).reshape(NW, EPW),
                       ((0, 0), (0, 2 * PCH * EC - EPW)))
    to_idx = jnp.pad(block_node_to_idx.astype(jnp.int32).reshape(NW, EPW),
                     ((0, 0), (0, 2 * PCH * EC - EPW)))

    wn = params['Wn']
    bn = params['bn'].reshape(1, D)
    we = params['We']
    be = params['be'].reshape(1, D)
    wg = params['Wg']
    bg = params['bg'].reshape(1, D)
    layers = params['layers']
    wmf = [lp['Wm'][:D] for lp in layers]
    wmt = [lp['Wm'][D:2 * D] for lp in layers]
    wme = [lp['Wm'][2 * D:] for lp in layers]
    bm = [lp['bm'].reshape(1, D) for lp in layers]
    wux = [lp['Wu'][:D] for lp in layers]
    wua = [lp['Wu'][D:] for lp in layers]
    bu = [lp['bu'].reshape(1, D) for lp in layers]

    gid3 = node_graph_idx.astype(jnp.int32).reshape(NBLK, 1, BN)
    cnt2 = node_num_per_g.reshape(G, 1)

    efm = _efm(block_edge_features.T, we, be, wme, bm)
    x, xf, xt = _encode(block_node_features, wn, bn, wmf[0], wmt[0])
    for l in range(2):
        acc = _edge_pass(xf, xt, efm[l], from_idx, to_idx)
        x, xf, xt = _update(x, acc, wux[l], wua[l], bu[l],
                            wmf[l + 1], wmt[l + 1])
    acc = _edge_pass(xf, xt, efm[2], from_idx, to_idx)
    return _update_pool(x, acc, wux[2], wua[2], bu[2], gid3, cnt2, wg, bg)
